# baseline (speedup 1.0000x reference)
import functools

import jax
import jax.numpy as jnp
from jax import lax
from jax.experimental import pallas as pl
from jax.experimental.pallas import tpu as pltpu
from jax.experimental.pallas import tpu_sc as plsc

D = 64
NSEQ = 4096
T = 200
SEGS = ((0, 104), (104, 96))
NW = 32
SEQ_PER_W = NSEQ // NW
S = 4
N_CHUNKS = SEQ_PER_W // S

_mesh = plsc.VectorSubcoreMesh(core_axis_name="c", subcore_axis_name="s")


@functools.partial(
    pl.kernel,
    mesh=_mesh,
    out_type=jax.ShapeDtypeStruct((NSEQ, T, D), jnp.float32),
    scratch_types=[
        pltpu.VMEM((S, T), jnp.int32),
        pltpu.VMEM((S, T), jnp.int32),
        pltpu.VMEM((S, T, D), jnp.float32),
        pltpu.VMEM((S, T, D), jnp.float32),
        pltpu.SemaphoreType.DMA,
        pltpu.SemaphoreType.DMA,
    ],
    compiler_params=pltpu.CompilerParams(use_tc_tiling_on_sc=False),
)
def _gather_kernel(table_hbm, idx_hbm, out_hbm, idx0, idx1, rows0, rows1,
                   gsem0, gsem1):
    wid = lax.axis_index("s") * 2 + lax.axis_index("c")
    seq_base = wid * SEQ_PER_W

    def fire(i, idx_buf, row_buf, sem):
        pltpu.sync_copy(idx_hbm.at[pl.ds(seq_base + i * S, S)], idx_buf)
        for s in range(S):
            for off, length in SEGS:
                pltpu.async_copy(
                    table_hbm.at[idx_buf.at[s, pl.ds(off, length)]],
                    row_buf.at[s, pl.ds(off, length)],
                    sem,
                )

    def drain(row_buf, sem):
        pltpu.make_async_copy(out_hbm.at[pl.ds(0, S)], row_buf, sem).wait()

    def writeback(row_buf, i):
        pltpu.sync_copy(row_buf, out_hbm.at[pl.ds(seq_base + i * S, S)])

    fire(0, idx0, rows0, gsem0)

    def body(t, _):
        a = 2 * t
        fire(a + 1, idx1, rows1, gsem1)
        drain(rows0, gsem0)
        writeback(rows0, a)

        @pl.when(a + 2 < N_CHUNKS)
        def _():
            fire(a + 2, idx0, rows0, gsem0)

        drain(rows1, gsem1)
        writeback(rows1, a + 1)
        return 0

    lax.fori_loop(0, N_CHUNKS // 2, body, 0)


def kernel(idx, embed_weight):
    return _gather_kernel(embed_weight, idx.astype(jnp.int32))
